# baseline (speedup 1.0000x reference)
import functools

import jax
import jax.numpy as jnp
from jax import lax
from jax.experimental import pallas as pl
from jax.experimental.pallas import tpu as pltpu
from jax.experimental.pallas import tpu_sc as plsc

_NC, _NS, _L = 2, 16, 16
_NW = _NC * _NS
_B = 16384
_BPW = _B // _NW
_D = 64
_R = 64
_G = _BPW // _L
_EPS = 1e-10
_LN2 = 0.6931471805599453
_SQRT2 = 1.4142135623730951


def _log16(x):
    ib = plsc.bitcast(x, jnp.int32)
    e = lax.shift_right_logical(ib, 23) - 127
    m = plsc.bitcast((ib & 0x007FFFFF) | 0x3F800000, jnp.float32)
    big = m > _SQRT2
    m = jnp.where(big, m * 0.5, m)
    e = jnp.where(big, e + 1, e)
    s = (m - 1.0) / (m + 1.0)
    s2 = s * s
    p = (((s2 * (1.0 / 9.0) + (1.0 / 7.0)) * s2 + (1.0 / 5.0)) * s2
         + (1.0 / 3.0)) * s2 + 1.0
    return e.astype(jnp.float32) * _LN2 + 2.0 * s * p


_mesh = plsc.VectorSubcoreMesh(core_axis_name="c", subcore_axis_name="s")


@functools.partial(
    pl.kernel,
    out_type=jax.ShapeDtypeStruct((_NW, _L), jnp.float32),
    mesh=_mesh,
    compiler_params=pltpu.CompilerParams(
        needs_layout_passes=False, use_tc_tiling_on_sc=False),
    scratch_types=[
        pltpu.VMEM((4, 128), jnp.int32),
        pltpu.VMEM((4, 128), jnp.int32),
        pltpu.VMEM((_BPW,), jnp.int32),
        pltpu.VMEM((_BPW,), jnp.float32),
        pltpu.VMEM((_BPW, _D), jnp.float32),
        pltpu.VMEM((_BPW, _D), jnp.float32),
        pltpu.VMEM((_R, _D), jnp.float32),
        pltpu.VMEM((_L,), jnp.float32),
        pltpu.SemaphoreType.DMA,
    ],
)
def _hin2vec_sc(a1_hbm, a2_hbm, rel_hbm, gt_hbm, emb_hbm, relemb_hbm, out_hbm,
                idx1_v, idx2_v, rel_v, gt_v, r1_v, r2_v, srel_v, lacc_v, sem):
    wid = lax.axis_index("s") * _NC + lax.axis_index("c")
    base = wid * _BPW
    row4 = wid * 4

    pltpu.sync_copy(a1_hbm.at[pl.ds(row4, 4)], idx1_v)
    pltpu.sync_copy(a2_hbm.at[pl.ds(row4, 4)], idx2_v)
    copies = []
    for j in range(4):
        copies.append(pltpu.async_copy(
            emb_hbm.at[idx1_v.at[j]], r1_v.at[pl.ds(j * 128, 128)], sem))
        copies.append(pltpu.async_copy(
            emb_hbm.at[idx2_v.at[j]], r2_v.at[pl.ds(j * 128, 128)], sem))

    pltpu.sync_copy(rel_hbm.at[pl.ds(base, _BPW)], rel_v)
    pltpu.sync_copy(gt_hbm.at[pl.ds(base, _BPW)], gt_v)
    pltpu.sync_copy(relemb_hbm, srel_v)

    def srel_body(r, carry):
        for k in range(_D // _L):
            x = srel_v[r, pl.ds(k * _L, _L)]
            srel_v[r, pl.ds(k * _L, _L)] = 1.0 / (1.0 + jnp.exp(-x))
        return carry
    lax.fori_loop(0, _R, srel_body, 0)

    for c in copies:
        c.wait()

    lane = lax.iota(jnp.int32, _L)

    def group_body(g, lacc):
        bvec = g * _L + lane
        rvec = rel_v[pl.ds(g * _L, _L)]
        gvec = gt_v[pl.ds(g * _L, _L)]
        acc = jnp.zeros((_L,), jnp.float32)
        for d in range(_D):
            dsp = jnp.full((_L,), d, jnp.int32)
            v1 = plsc.load_gather(r1_v, [bvec, dsp])
            v2 = plsc.load_gather(r2_v, [bvec, dsp])
            vs = plsc.load_gather(srel_v, [rvec, dsp])
            acc = acc + v1 * v2 * vs
        pred = 1.0 / (1.0 + jnp.exp(-acc))
        lp = _log16(pred + _EPS)
        lq = _log16(1.0 - pred + _EPS)
        return lacc - (gvec * lp + (1.0 - gvec) * lq)

    lacc = lax.fori_loop(0, _G, group_body, jnp.zeros((_L,), jnp.float32))
    lacc_v[...] = lacc
    pltpu.sync_copy(lacc_v, out_hbm.at[wid])


def kernel(attr1, attr2, rel, ground_truth, embeddings, relation_embedding):
    a1 = attr1.astype(jnp.int32).reshape(128, 128)
    a2 = attr2.astype(jnp.int32).reshape(128, 128)
    r = rel.astype(jnp.int32)
    partials = _hin2vec_sc(a1, a2, r, ground_truth, embeddings,
                           relation_embedding)
    return jnp.sum(partials)


# speedup vs baseline: 1.7320x; 1.7320x over previous
import functools

import jax
import jax.numpy as jnp
from jax import lax
from jax.experimental import pallas as pl
from jax.experimental.pallas import tpu as pltpu
from jax.experimental.pallas import tpu_sc as plsc

_NC, _NS = 2, 16
_NW = _NC * _NS
_B = 16384
_BPW = _B // _NW
_D = 64
_EPS = 1e-10

_mesh = plsc.VectorSubcoreMesh(core_axis_name="c", subcore_axis_name="s")


@functools.partial(
    pl.kernel,
    out_type=[jax.ShapeDtypeStruct((_B, _D), jnp.float32),
              jax.ShapeDtypeStruct((_B, _D), jnp.float32)],
    mesh=_mesh,
    compiler_params=pltpu.CompilerParams(
        needs_layout_passes=False, use_tc_tiling_on_sc=True),
    scratch_types=[
        pltpu.VMEM((_BPW,), jnp.int32),
        pltpu.VMEM((_BPW, _D), jnp.float32),
        pltpu.SemaphoreType.DMA,
    ],
)
def _gather_sc(a1_hbm, a2_hbm, emb_hbm, o1_hbm, o2_hbm,
               idx_v, rows_v, sem):
    wid = lax.axis_index("s") * _NC + lax.axis_index("c")
    base = wid * _BPW
    for src, dst in ((a1_hbm, o1_hbm), (a2_hbm, o2_hbm)):
        pltpu.sync_copy(src.at[pl.ds(base, _BPW)], idx_v)

        def group_body(g, carry):
            vec = idx_v[pl.ds(g * 16, 16)]
            for k in range(16):
                r = vec[k]
                pltpu.async_copy(
                    emb_hbm.at[pl.ds(r, 1)],
                    rows_v.at[pl.ds(g * 16 + k, 1)], sem)
            return carry

        lax.fori_loop(0, _BPW // 16, group_body, 0)
        pltpu.make_async_copy(
            emb_hbm.at[pl.ds(0, _BPW)], rows_v, sem).wait()
        pltpu.sync_copy(rows_v, dst.at[pl.ds(base, _BPW)])


_BB = 2048
_NB = _B // _BB


def _loss_tc(e1_ref, e2_ref, rel_ref, gt_ref, w_ref, out_ref):
    i = pl.program_id(0)
    w = jax.nn.sigmoid(w_ref[...])
    p = e1_ref[...] * e2_ref[...]
    s = lax.dot_general(p, w, (((1,), (1,)), ((), ())),
                        preferred_element_type=jnp.float32)
    col = lax.broadcasted_iota(jnp.int32, s.shape, 1)
    acc = jnp.sum(jnp.where(col == rel_ref[...], s, 0.0),
                  axis=1, keepdims=True)
    pred = jax.nn.sigmoid(acc)
    gt = gt_ref[...]
    loss = -(gt * jnp.log(pred + _EPS)
             + (1.0 - gt) * jnp.log(1.0 - pred + _EPS))
    part = jnp.sum(loss, keepdims=True).reshape(1, 1)

    @pl.when(i == 0)
    def _init():
        out_ref[...] = part

    @pl.when(i != 0)
    def _acc():
        out_ref[...] += part


def kernel(attr1, attr2, rel, ground_truth, embeddings, relation_embedding):
    a1 = attr1.astype(jnp.int32)
    a2 = attr2.astype(jnp.int32)
    e1, e2 = _gather_sc(a1, a2, embeddings)
    rel2 = rel.astype(jnp.int32).reshape(_B, 1)
    gt2 = ground_truth.reshape(_B, 1)
    out = pl.pallas_call(
        _loss_tc,
        grid=(_NB,),
        in_specs=[
            pl.BlockSpec((_BB, _D), lambda i: (i, 0)),
            pl.BlockSpec((_BB, _D), lambda i: (i, 0)),
            pl.BlockSpec((_BB, 1), lambda i: (i, 0)),
            pl.BlockSpec((_BB, 1), lambda i: (i, 0)),
            pl.BlockSpec((_D, _D), lambda i: (0, 0)),
        ],
        out_specs=pl.BlockSpec((1, 1), lambda i: (0, 0)),
        out_shape=jax.ShapeDtypeStruct((1, 1), jnp.float32),
    )(e1, e2, rel2, gt2, relation_embedding)
    return out[0, 0]
